# baseline (speedup 1.0000x reference)
import functools

import jax
import jax.numpy as jnp
from jax import lax
from jax.experimental import pallas as pl
from jax.experimental.pallas import tpu as pltpu
from jax.experimental.pallas import tpu_sc as plsc

B, L = 16384, 200
N = B * L
V = 1_000_000
VP = 1 << 20
NC, NS = 2, 16
NW = NC * NS

CPW = N // NW
C = 2048
NCHUNK = CPW // C
SW = 256
G = C // SW
VSL = VP // NS
VLAST = V - 15 * VSL


@functools.partial(
    pl.kernel,
    out_type=jax.ShapeDtypeStruct((3 * N,), jnp.float32),
    mesh=plsc.VectorSubcoreMesh(core_axis_name="c", subcore_axis_name="s"),
    compiler_params=pltpu.CompilerParams(
        needs_layout_passes=False, use_tc_tiling_on_sc=False
    ),
    scratch_types=[
        pltpu.VMEM_SHARED((V,), jnp.float32),
        pltpu.VMEM((16,), jnp.float32),
        pltpu.VMEM((2, C), jnp.int32),
        pltpu.VMEM((2, C), jnp.float32),
        pltpu.VMEM((2, 3 * C), jnp.float32),
        pltpu.SemaphoreType.DMA,
        pltpu.SemaphoreType.DMA,
        pltpu.SemaphoreType.DMA,
        pltpu.SemaphoreType.DMA,
        pltpu.SemaphoreType.DMA,
    ],
)
def _fused(tab_hbm, w_hbm, b_hbm, idx_hbm, out_hbm,
           tab_sp, wb_v, idx_v, sv_v, out3_v,
           sem_i0, sem_i1, sem_g, sem_o0, sem_o1):
    cid = lax.axis_index("c")
    sid = lax.axis_index("s")
    wid = sid * NC + cid

    @pl.when(sid < 15)
    def _():
        pltpu.sync_copy(
            tab_hbm.at[pl.ds(sid * VSL, VSL)], tab_sp.at[pl.ds(sid * VSL, VSL)]
        )

    @pl.when(sid == 15)
    def _():
        pltpu.sync_copy(
            tab_hbm.at[pl.ds(15 * VSL, VLAST)],
            tab_sp.at[pl.ds(15 * VSL, VLAST)],
        )

    pltpu.sync_copy(w_hbm, wb_v.at[pl.ds(0, 3)])
    pltpu.sync_copy(b_hbm, wb_v.at[pl.ds(8, 3)])
    plsc.subcore_barrier()

    ii = lax.broadcasted_iota(jnp.int32, (16,), 0)
    npat, wpat, bpat = [], [], []
    for r in range(3):
        m = ii + 16 * r
        kk = m % 3
        npat.append(m // 3)
        wpat.append(plsc.load_gather(wb_v, [kk]))
        bpat.append(plsc.load_gather(wb_v, [kk + 8]))

    base0 = wid * CPW
    sem_i = (sem_i0, sem_i1)
    sem_o = (sem_o0, sem_o1)

    def idx_src(u):
        return idx_hbm.at[pl.ds(base0 + u * C, C)]

    def out_dst(u):
        return out_hbm.at[pl.ds(3 * (base0 + u * C), 3 * C)]

    def gathers(b):
        return [
            pltpu.async_copy(
                tab_sp.at[idx_v.at[b, pl.ds(g * SW, SW)]],
                sv_v.at[b, pl.ds(g * SW, SW)],
                sem_g,
            )
            for g in range(G)
        ]

    def expand(b):
        @pl.loop(0, C // 16)
        def jloop(j):
            nb = j * 16
            for r in range(3):
                v = plsc.load_gather(sv_v.at[b], [npat[r] + nb])
                out3_v[b, pl.ds(j * 48 + r * 16, 16)] = v * wpat[r] + bpat[r]

    pltpu.async_copy(idx_src(0), idx_v.at[0], sem_i0)
    pltpu.async_copy(idx_src(1), idx_v.at[1], sem_i1)

    for b in (0, 1):
        pltpu.make_async_copy(idx_src(b), idx_v.at[b], sem_i[b]).wait()
        for cp in gathers(b):
            cp.wait()
        pltpu.async_copy(idx_src(b + 2), idx_v.at[b], sem_i[b])
        expand(b)
        pltpu.async_copy(out3_v.at[b], out_dst(b), sem_o[b])

    @pl.loop(1, NCHUNK // 2 - 1)
    def pair(t):
        for b in (0, 1):
            u = 2 * t + b
            pltpu.make_async_copy(idx_src(u), idx_v.at[b], sem_i[b]).wait()
            for cp in gathers(b):
                cp.wait()

            @pl.when(u + 2 < NCHUNK)
            def _():
                pltpu.async_copy(idx_src(u + 2), idx_v.at[b], sem_i[b])

            pltpu.make_async_copy(out3_v.at[b], out_dst(u), sem_o[b]).wait()
            expand(b)
            pltpu.async_copy(out3_v.at[b], out_dst(u), sem_o[b])

    for b in (0, 1):
        u = NCHUNK - 2 + b
        pltpu.make_async_copy(idx_src(u), idx_v.at[b], sem_i[b]).wait()
        for cp in gathers(b):
            cp.wait()
        pltpu.make_async_copy(out3_v.at[b], out_dst(u), sem_o[b]).wait()
        expand(b)
        pltpu.sync_copy(out3_v.at[b], out_dst(u))


def kernel(data, table, W, b):
    idx = data.reshape(-1)
    tab = table.reshape(-1)
    out = _fused(tab, W.reshape(-1), b.reshape(-1), idx)
    return out.reshape(B, L, 3)


# speedup vs baseline: 1.3126x; 1.3126x over previous
import functools

import jax
import jax.numpy as jnp
from jax import lax
from jax.experimental import pallas as pl
from jax.experimental.pallas import tpu as pltpu
from jax.experimental.pallas import tpu_sc as plsc

B, L = 16384, 200
N = B * L
V = 1_000_000
VP = 1 << 20
NC, NS = 2, 16
NW = NC * NS

CPW = N // NW
RPW = CPW // L
CR = 8
C = CR * L
NCHUNK = RPW // CR
SW = 200
G = C // SW
VSL = VP // NS
VLAST = V - 15 * VSL


@functools.partial(
    pl.kernel,
    out_type=jax.ShapeDtypeStruct((B, L, 3), jnp.float32),
    mesh=plsc.VectorSubcoreMesh(core_axis_name="c", subcore_axis_name="s"),
    compiler_params=pltpu.CompilerParams(
        needs_layout_passes=False, use_tc_tiling_on_sc=False
    ),
    scratch_types=[
        pltpu.VMEM_SHARED((V,), jnp.float32),
        pltpu.VMEM((16,), jnp.float32),
        pltpu.VMEM((2, C), jnp.int32),
        pltpu.VMEM((2, C), jnp.float32),
        pltpu.VMEM((2, CR, L, 3), jnp.float32),
        pltpu.SemaphoreType.DMA,
        pltpu.SemaphoreType.DMA,
        pltpu.SemaphoreType.DMA,
        pltpu.SemaphoreType.DMA,
        pltpu.SemaphoreType.DMA,
    ],
)
def _fused(tab_hbm, w_hbm, b_hbm, idx_hbm, out_hbm,
           tab_sp, wb_v, idx_v, sv_v, out3_v,
           sem_i0, sem_i1, sem_g, sem_o0, sem_o1):
    cid = lax.axis_index("c")
    sid = lax.axis_index("s")
    wid = sid * NC + cid

    @pl.when(sid < 15)
    def _():
        pltpu.sync_copy(
            tab_hbm.at[pl.ds(sid * VSL, VSL)], tab_sp.at[pl.ds(sid * VSL, VSL)]
        )

    @pl.when(sid == 15)
    def _():
        pltpu.sync_copy(
            tab_hbm.at[pl.ds(15 * VSL, VLAST)],
            tab_sp.at[pl.ds(15 * VSL, VLAST)],
        )

    pltpu.sync_copy(w_hbm, wb_v.at[pl.ds(0, 3)])
    pltpu.sync_copy(b_hbm, wb_v.at[pl.ds(8, 3)])
    plsc.subcore_barrier()

    ii = lax.broadcasted_iota(jnp.int32, (16,), 0)
    npat, wpat, bpat = [], [], []
    for r in range(3):
        m = ii + 16 * r
        kk = m % 3
        npat.append(m // 3)
        wpat.append(plsc.load_gather(wb_v, [kk]))
        bpat.append(plsc.load_gather(wb_v, [kk + 8]))

    base0 = wid * CPW
    sem_i = (sem_i0, sem_i1)
    sem_o = (sem_o0, sem_o1)

    def idx_src(u):
        return idx_hbm.at[pl.ds(base0 + u * C, C)]

    rbase0 = wid * RPW

    def out_dst(u):
        return out_hbm.at[pl.ds(rbase0 + u * CR, CR)]

    def gathers(b):
        return [
            pltpu.async_copy(
                tab_sp.at[idx_v.at[b, pl.ds(g * SW, SW)]],
                sv_v.at[b, pl.ds(g * SW, SW)],
                sem_g,
            )
            for g in range(G)
        ]

    def expand(b):
        @pl.loop(0, 3 * C // 16)
        def jloop(j):
            m = j * 16 + ii
            row = m // 600
            rem = m % 600
            nn = rem // 3
            kk = rem % 3
            v = plsc.load_gather(sv_v.at[b], [m // 3])
            w = plsc.load_gather(wb_v, [kk])
            bb = plsc.load_gather(wb_v, [kk + 8])
            plsc.store_scatter(out3_v.at[b], [row, nn, kk], v * w + bb)

    pltpu.async_copy(idx_src(0), idx_v.at[0], sem_i0)
    pltpu.async_copy(idx_src(1), idx_v.at[1], sem_i1)

    for b in (0, 1):
        pltpu.make_async_copy(idx_src(b), idx_v.at[b], sem_i[b]).wait()
        for cp in gathers(b):
            cp.wait()
        pltpu.async_copy(idx_src(b + 2), idx_v.at[b], sem_i[b])
        expand(b)
        pltpu.async_copy(out3_v.at[b], out_dst(b), sem_o[b])

    @pl.loop(1, NCHUNK // 2 - 1)
    def pair(t):
        for b in (0, 1):
            u = 2 * t + b
            pltpu.make_async_copy(idx_src(u), idx_v.at[b], sem_i[b]).wait()
            for cp in gathers(b):
                cp.wait()

            @pl.when(u + 2 < NCHUNK)
            def _():
                pltpu.async_copy(idx_src(u + 2), idx_v.at[b], sem_i[b])

            pltpu.make_async_copy(out3_v.at[b], out_dst(u), sem_o[b]).wait()
            expand(b)
            pltpu.async_copy(out3_v.at[b], out_dst(u), sem_o[b])

    for b in (0, 1):
        u = NCHUNK - 2 + b
        pltpu.make_async_copy(idx_src(u), idx_v.at[b], sem_i[b]).wait()
        for cp in gathers(b):
            cp.wait()
        pltpu.make_async_copy(out3_v.at[b], out_dst(u), sem_o[b]).wait()
        expand(b)
        pltpu.sync_copy(out3_v.at[b], out_dst(u))


def kernel(data, table, W, b):
    idx = data.reshape(-1)
    tab = table.reshape(-1)
    out = _fused(tab, W.reshape(-1), b.reshape(-1), idx)
    return out.reshape(B, L, 3)
